# baseline (speedup 1.0000x reference)
import jax
import jax.numpy as jnp
from jax import lax
from jax.experimental import pallas as pl
from jax.experimental.pallas import tpu as pltpu
from jax.experimental.pallas import tpu_sc as plsc

B = 16384
F = 32
NC = 2
NS = 16
NW = NC * NS
BPW = B // NW
NCHUNK = 4
CH = BPW // NCHUNK
L = 16


def _gmf_body(user_hbm, item_hbm, eu_hbm, ei_hbm, w_hbm, b_hbm, out_hbm,
              idx_u, idx_i, rows_u, rows_i, out_v, w_v, b_v, sem):
    wid = lax.axis_index("s") * NC + lax.axis_index("c")
    base = wid * BPW

    for j in range(NCHUNK):
        pltpu.sync_copy(user_hbm.at[pl.ds(base + j * CH, CH)], idx_u.at[j])
        pltpu.sync_copy(item_hbm.at[pl.ds(base + j * CH, CH)], idx_i.at[j])
    pltpu.sync_copy(w_hbm, w_v)
    pltpu.sync_copy(b_hbm, b_v)

    copies = []
    for j in range(NCHUNK):
        copies.append(pltpu.async_copy(eu_hbm.at[idx_u.at[j]], rows_u.at[j], sem))
        copies.append(pltpu.async_copy(ei_hbm.at[idx_i.at[j]], rows_i.at[j], sem))
    for c in copies:
        c.wait()

    w_lo = w_v[pl.ds(0, L)]
    w_hi = w_v[pl.ds(L, L)]
    bv = b_v[...]
    bsc = bv[0]
    lanes = lax.iota(jnp.int32, L)

    grp_per_chunk = CH // L

    def group(g, carry):
        j = g // grp_per_chunk
        r0 = (g % grp_per_chunk) * L
        acc = bv
        for i in range(L):
            r = r0 + i
            t = (rows_u[j, r, pl.ds(0, L)] * rows_i[j, r, pl.ds(0, L)] * w_lo
                 + rows_u[j, r, pl.ds(L, L)] * rows_i[j, r, pl.ds(L, L)] * w_hi)
            s = jnp.sum(t) + bsc
            acc = jnp.where(lanes == i, s, acc)
        out_v[pl.ds(g * L, L)] = acc
        return carry

    lax.fori_loop(0, BPW // L, group, 0)

    pltpu.sync_copy(out_v, out_hbm.at[pl.ds(base, BPW)])


@jax.jit
def kernel(user, item, embed_user, embed_item, W, b):
    mesh = plsc.VectorSubcoreMesh(core_axis_name="c", subcore_axis_name="s")
    kern = pl.kernel(
        _gmf_body,
        out_type=jax.ShapeDtypeStruct((B,), jnp.float32),
        mesh=mesh,
        compiler_params=pltpu.CompilerParams(
            needs_layout_passes=False, use_tc_tiling_on_sc=False),
        scratch_types=[
            pltpu.VMEM((NCHUNK, CH), jnp.int32),
            pltpu.VMEM((NCHUNK, CH), jnp.int32),
            pltpu.VMEM((NCHUNK, CH, F), jnp.float32),
            pltpu.VMEM((NCHUNK, CH, F), jnp.float32),
            pltpu.VMEM((BPW,), jnp.float32),
            pltpu.VMEM((F,), jnp.float32),
            pltpu.VMEM((L,), jnp.float32),
            pltpu.SemaphoreType.DMA,
        ],
    )
    return kern(user.astype(jnp.int32), item.astype(jnp.int32),
                embed_user, embed_item, W.reshape(F),
                jnp.full((L,), b[0], dtype=jnp.float32))
